# speedup vs baseline: 1.5063x; 1.5063x over previous
import functools

import jax
import jax.numpy as jnp
from jax.experimental import pallas as pl
from jax.experimental.pallas import tpu as pltpu

N = 524288
F = 128
P = 16
BLOCK = 4096


def _match_block(x_ref, th_ref, pat_ref, o_ref):
    pat = pat_ref[...]
    col = jax.lax.broadcasted_iota(jnp.int32, (P, F), 1)
    mask = (pat[:, None] == col).any(axis=0)
    x = x_ref[...]
    th = th_ref[...]
    ok = (x > th[None, :]) | (~mask)[None, :]
    o_ref[...] = ok.all(axis=1).astype(jnp.float32)


@jax.jit
def kernel(inputs, thresholds, pat_index):
    out = pl.pallas_call(
        _match_block,
        grid=(N // BLOCK,),
        in_specs=[
            pl.BlockSpec((BLOCK, F), lambda i: (i, 0)),
            pl.BlockSpec((F,), lambda i: (0,)),
            pl.BlockSpec((P,), lambda i: (0,)),
        ],
        out_specs=pl.BlockSpec((BLOCK,), lambda i: (i,)),
        out_shape=jax.ShapeDtypeStruct((N,), jnp.float32),
    )(inputs, thresholds, pat_index)
    return out.astype(jnp.bool_)


# speedup vs baseline: 1.5753x; 1.0458x over previous
import functools

import jax
import jax.numpy as jnp
from jax.experimental import pallas as pl
from jax.experimental.pallas import tpu as pltpu

N = 524288
F = 128
P = 16
BLOCK = 8192


def _match_block(x_ref, th_ref, pat_ref, o_ref):
    pat = pat_ref[...]
    col = jax.lax.broadcasted_iota(jnp.int32, (P, F), 1)
    mask = (pat[:, None] == col).any(axis=0)
    x = x_ref[...]
    th = th_ref[...]
    ok = (x > th[None, :]) | (~mask)[None, :]
    o_ref[...] = ok.all(axis=1).astype(jnp.float32)


@jax.jit
def kernel(inputs, thresholds, pat_index):
    out = pl.pallas_call(
        _match_block,
        grid=(N // BLOCK,),
        in_specs=[
            pl.BlockSpec((BLOCK, F), lambda i: (i, 0)),
            pl.BlockSpec((F,), lambda i: (0,)),
            pl.BlockSpec((P,), lambda i: (0,)),
        ],
        out_specs=pl.BlockSpec((BLOCK,), lambda i: (i,)),
        out_shape=jax.ShapeDtypeStruct((N,), jnp.float32),
    )(inputs, thresholds, pat_index)
    return out.astype(jnp.bool_)


# speedup vs baseline: 1.6698x; 1.0599x over previous
import functools

import jax
import jax.numpy as jnp
from jax.experimental import pallas as pl
from jax.experimental.pallas import tpu as pltpu

N = 524288
F = 128
P = 16
BLOCK = 8192


def _match_block(x_ref, th_ref, pat_ref, o_ref):
    pat = pat_ref[...]
    col = jax.lax.broadcasted_iota(jnp.int32, (P, F), 1)
    mask = (pat[:, None] == col).any(axis=0)
    x = x_ref[...]
    th = th_ref[...]
    ok = ((x > th[None, :]) | (~mask)[None, :]).astype(jnp.float32)
    cnt = jax.lax.dot_general(
        ok, jnp.ones((F, 1), jnp.float32),
        dimension_numbers=(((1,), (0,)), ((), ())),
        preferred_element_type=jnp.float32,
    )
    o_ref[...] = (cnt[:, 0] == jnp.float32(F)).astype(jnp.float32)


@jax.jit
def kernel(inputs, thresholds, pat_index):
    out = pl.pallas_call(
        _match_block,
        grid=(N // BLOCK,),
        in_specs=[
            pl.BlockSpec((BLOCK, F), lambda i: (i, 0)),
            pl.BlockSpec((F,), lambda i: (0,)),
            pl.BlockSpec((P,), lambda i: (0,)),
        ],
        out_specs=pl.BlockSpec((BLOCK,), lambda i: (i,)),
        out_shape=jax.ShapeDtypeStruct((N,), jnp.float32),
    )(inputs, thresholds, pat_index)
    return out.astype(jnp.bool_)


# speedup vs baseline: 3.7866x; 2.2677x over previous
import functools

import jax
import jax.numpy as jnp
from jax.experimental import pallas as pl
from jax.experimental.pallas import tpu as pltpu

N = 524288
F = 128
P = 16
BLOCK = 8192


def _match_block(x_ref, th_ref, pat_ref, o_ref):
    pat = pat_ref[...]
    col = jax.lax.broadcasted_iota(jnp.int32, (P, F), 1)
    mask = (pat[:, None] == col).any(axis=0)
    x = x_ref[...]
    th = th_ref[...]
    fail = ((x <= th[None, :]) & mask[None, :]).astype(jnp.float32)
    cnt = jax.lax.dot_general(
        jnp.ones((1, F), jnp.float32), fail,
        dimension_numbers=(((1,), (1,)), ((), ())),
        preferred_element_type=jnp.float32,
    )
    o_ref[...] = (cnt == 0.0).astype(jnp.float32)[None]


@jax.jit
def kernel(inputs, thresholds, pat_index):
    out = pl.pallas_call(
        _match_block,
        grid=(N // BLOCK,),
        in_specs=[
            pl.BlockSpec((BLOCK, F), lambda i: (i, 0)),
            pl.BlockSpec((F,), lambda i: (0,)),
            pl.BlockSpec((P,), lambda i: (0,)),
        ],
        out_specs=pl.BlockSpec((1, 1, BLOCK), lambda i: (i, 0, 0)),
        out_shape=jax.ShapeDtypeStruct((N // BLOCK, 1, BLOCK), jnp.float32),
    )(inputs, thresholds, pat_index)
    return out.reshape(N).astype(jnp.bool_)


# speedup vs baseline: 4.5110x; 1.1913x over previous
import functools

import jax
import jax.numpy as jnp
from jax.experimental import pallas as pl
from jax.experimental.pallas import tpu as pltpu

N = 524288
F = 128
P = 16
BLOCK = 16384


def _match_block(x_ref, th_ref, pat_ref, o_ref):
    pat = pat_ref[...]
    col = jax.lax.broadcasted_iota(jnp.int32, (P, F), 1)
    mask = (pat[:, None] == col).any(axis=0)
    x = x_ref[...]
    th = th_ref[...]
    fail = ((x <= th[None, :]) & mask[None, :]).astype(jnp.float32)
    cnt = jax.lax.dot_general(
        jnp.ones((1, F), jnp.float32), fail,
        dimension_numbers=(((1,), (1,)), ((), ())),
        preferred_element_type=jnp.float32,
    )
    o_ref[...] = (cnt == 0.0).astype(jnp.float32)[None]


@jax.jit
def kernel(inputs, thresholds, pat_index):
    out = pl.pallas_call(
        _match_block,
        grid=(N // BLOCK,),
        in_specs=[
            pl.BlockSpec((BLOCK, F), lambda i: (i, 0)),
            pl.BlockSpec((F,), lambda i: (0,)),
            pl.BlockSpec((P,), lambda i: (0,)),
        ],
        out_specs=pl.BlockSpec((1, 1, BLOCK), lambda i: (i, 0, 0)),
        out_shape=jax.ShapeDtypeStruct((N // BLOCK, 1, BLOCK), jnp.float32),
    )(inputs, thresholds, pat_index)
    return out.reshape(N).astype(jnp.bool_)


# speedup vs baseline: 4.8260x; 1.0698x over previous
import functools

import jax
import jax.numpy as jnp
from jax.experimental import pallas as pl
from jax.experimental.pallas import tpu as pltpu

N = 524288
F = 128
P = 16
BLOCK = 32768


def _match_block(x_ref, th_ref, pat_ref, o_ref):
    pat = pat_ref[...]
    col = jax.lax.broadcasted_iota(jnp.int32, (P, F), 1)
    mask = (pat[:, None] == col).any(axis=0)
    x = x_ref[...]
    th = th_ref[...]
    fail = ((x <= th[None, :]) & mask[None, :]).astype(jnp.float32)
    cnt = jax.lax.dot_general(
        jnp.ones((1, F), jnp.float32), fail,
        dimension_numbers=(((1,), (1,)), ((), ())),
        preferred_element_type=jnp.float32,
    )
    o_ref[...] = (cnt == 0.0).astype(jnp.float32)[None]


@jax.jit
def kernel(inputs, thresholds, pat_index):
    out = pl.pallas_call(
        _match_block,
        grid=(N // BLOCK,),
        in_specs=[
            pl.BlockSpec((BLOCK, F), lambda i: (i, 0)),
            pl.BlockSpec((F,), lambda i: (0,)),
            pl.BlockSpec((P,), lambda i: (0,)),
        ],
        out_specs=pl.BlockSpec((1, 1, BLOCK), lambda i: (i, 0, 0)),
        out_shape=jax.ShapeDtypeStruct((N // BLOCK, 1, BLOCK), jnp.float32),
    )(inputs, thresholds, pat_index)
    return out.reshape(N).astype(jnp.bool_)
